# baseline (speedup 1.0000x reference)
import jax
import jax.numpy as jnp
from jax.experimental import pallas as pl
from jax.experimental.pallas import tpu as pltpu

_ROW_MAP = ((0, 0), (6, 1), (1, 2))


def _gather_body(slab_hbm, out_hbm, sem):
    copies = [
        pltpu.make_async_copy(slab_hbm.at[0, pl.ds(2, 3)], out_hbm.at[0], sem)
    ] + [
        pltpu.make_async_copy(
            slab_hbm.at[1, pl.ds(j, 1)], out_hbm.at[1, pl.ds(b, 1)], sem
        )
        for j, b in _ROW_MAP
    ]
    for c in copies:
        c.start()
    for c in copies:
        c.wait()


def kernel(x):
    slab = jax.lax.slice(x, (0, 0, 0), (2, 8, 128))
    return pl.pallas_call(
        _gather_body,
        in_specs=[pl.BlockSpec(memory_space=pl.ANY)],
        out_specs=pl.BlockSpec(memory_space=pl.ANY),
        out_shape=jax.ShapeDtypeStruct((2, 3, 128), jnp.float32),
        scratch_shapes=[pltpu.SemaphoreType.DMA],
    )(slab)
